# baseline (speedup 1.0000x reference)
import functools

import jax
import jax.numpy as jnp
from jax import lax
from jax.experimental import pallas as pl
from jax.experimental.pallas import tpu as pltpu
from jax.experimental.pallas import tpu_sc as plsc

B, L, D = 128, 512, 768
N = B * L
NC, NS, LANES = 2, 16, 16
NW = NC * NS
Q = 4
DQ = D // Q
TB = NW // Q
TPB = N // TB
C = LANES
NCHUNK = TPB // C
NBUF = 4
KV = DQ // LANES


def _pack_body(tok, pos, seg, packed):
    packed[...] = tok[...] | ((seg[...] * 512 + pos[...]) << 15)


@jax.jit
def _pack(tok, pos, seg):
    return pl.pallas_call(
        _pack_body,
        out_shape=jax.ShapeDtypeStruct((B, L), jnp.int32),
    )(tok, pos, seg)


def _sc_body(packed_hbm, ttab4, loc4, out4, idx_buf, loc_loc, pbuf, sbuf, *rest):
    bufT = rest[0:NBUF]
    sem_in = rest[NBUF:2 * NBUF]
    sem_out = rest[2 * NBUF:3 * NBUF]

    wid = lax.axis_index("s") * NC + lax.axis_index("c")
    q = wid % Q
    tok_base = (wid // Q) * TPB

    pltpu.sync_copy(loc4.at[q], loc_loc)
    pltpu.sync_copy(packed_hbm.at[pl.ds(tok_base, TPB)], idx_buf)

    iota = lax.broadcasted_iota(jnp.int32, (LANES,), 0)

    def fire_in(cg, b):
        pk = idx_buf[pl.ds(cg * C, C)]
        gvec = ((pk & 0x7FFF) << 2) + q
        pltpu.async_copy(ttab4.at[gvec], bufT[b], sem_in[b])

    def drain_in(b):
        pltpu.make_async_copy(ttab4.at[pl.ds(0, C)], bufT[b], sem_in[b]).wait()

    def fire_out(cg, b):
        ovec = ((tok_base + cg * C + iota) << 2) + q
        pltpu.async_copy(bufT[b], out4.at[ovec], sem_out[b])

    def drain_out(b):
        pltpu.make_async_copy(bufT[b], out4.at[pl.ds(0, C)], sem_out[b]).wait()

    fire_in(0, 0)
    fire_in(1, 1)

    def step(qq, carry):
        for b in range(NBUF):
            cg = qq * NBUF + b
            drain_in(b)

            pk = idx_buf[pl.ds(cg * C, C)]
            pbuf[...] = ((pk >> 15) & 511) * DQ
            sbuf[...] = ((pk >> 24) + 512) * DQ

            def group(g, carry2):
                for jj in range(8):
                    j = g * 8 + jj
                    jf = jnp.full((LANES,), j, jnp.int32)
                    rp = plsc.load_gather(pbuf, [jf])
                    rs = plsc.load_gather(sbuf, [jf])
                    for k in range(KV):
                        col = iota + (k * LANES)
                        g1 = plsc.load_gather(loc_loc, [rp + col])
                        g2 = plsc.load_gather(loc_loc, [rs + col])
                        plsc.addupdate(bufT[b].at[j, pl.ds(k * LANES, LANES)],
                                       g1 + g2)
                return carry2

            lax.fori_loop(0, C // 8, group, 0)
            fire_out(cg, b)

            b2 = (b + 2) % NBUF

            @pl.when(cg >= 2)
            def _():
                drain_out(b2)

            @pl.when(cg + 2 < NCHUNK)
            def _():
                fire_in(cg + 2, b2)
        return carry

    lax.fori_loop(0, NCHUNK // NBUF, step, 0)

    drain_out((NCHUNK - 2) % NBUF)
    drain_out((NCHUNK - 1) % NBUF)


@jax.jit
def _embed_sum(packed, ttab4, loc4):
    mesh = plsc.VectorSubcoreMesh(core_axis_name="c", subcore_axis_name="s")
    scratch = [
        pltpu.VMEM((TPB,), jnp.int32),
        pltpu.VMEM(((512 + 2) * DQ,), jnp.float32),
        pltpu.VMEM((LANES,), jnp.int32),
        pltpu.VMEM((LANES,), jnp.int32),
    ]
    scratch += [pltpu.VMEM((C, DQ), jnp.float32) for _ in range(NBUF)]
    scratch += [pltpu.SemaphoreType.DMA for _ in range(2 * NBUF)]
    f = functools.partial(
        pl.kernel,
        mesh=mesh,
        out_type=jax.ShapeDtypeStruct((N * Q, DQ), jnp.float32),
        scratch_types=scratch,
        compiler_params=pltpu.CompilerParams(
            use_tc_tiling_on_sc=False, needs_layout_passes=False),
    )(_sc_body)
    return f(packed, ttab4, loc4)


def kernel(token_ids, position_ids, segment_ids, token_table, position_table, segment_table):
    packed = _pack(token_ids.astype(jnp.int32), position_ids.astype(jnp.int32),
                   segment_ids.astype(jnp.int32))
    ttab4 = token_table.reshape(30522 * Q, DQ)
    loc4 = jnp.transpose(
        jnp.concatenate([position_table, segment_table], axis=0)
        .reshape(512 + 2, Q, DQ), (1, 0, 2)).reshape(Q, (512 + 2) * DQ)
    out = _embed_sum(packed.reshape(N), ttab4, loc4)
    return out.reshape(B, L, D)


# speedup vs baseline: 3.5672x; 3.5672x over previous
import functools

import jax
import jax.numpy as jnp
from jax import lax
from jax.experimental import pallas as pl
from jax.experimental.pallas import tpu as pltpu
from jax.experimental.pallas import tpu_sc as plsc

B, L, D = 128, 512, 768
N = B * L
NC, NS, LANES = 2, 16, 16
NW = NC * NS
PER_W = N // NW
C = LANES
NCHUNK = PER_W // C
NBUF = 2
DV = D // LANES


def _prep_body(ptab, stab, pos, seg, comb, cid):
    p = ptab[...]
    comb[pl.ds(0, 512), :] = p + stab[0:1, :]
    comb[pl.ds(512, 512), :] = p + stab[1:2, :]
    cid[...] = seg[...] * 512 + pos[...]


@jax.jit
def _prep(ptab, stab, pos, seg):
    return pl.pallas_call(
        _prep_body,
        out_shape=(
            jax.ShapeDtypeStruct((2 * 512, D), jnp.float32),
            jax.ShapeDtypeStruct((B, L), jnp.int32),
        ),
    )(ptab, stab, pos, seg)


def _sc_body(tok_hbm, cid_hbm, ttab, ctab, out_hbm, *scratch):
    tok_idx, cid_idx = scratch[0], scratch[1]
    bufT = scratch[2:2 + NBUF]
    bufC = scratch[2 + NBUF:2 + 2 * NBUF]
    bufO = scratch[2 + 2 * NBUF:2 + 3 * NBUF]
    sem_in = scratch[2 + 3 * NBUF:2 + 4 * NBUF]
    sem_out = scratch[2 + 4 * NBUF:2 + 5 * NBUF]

    sid = lax.axis_index("s")
    wid = sid * NC + lax.axis_index("c")
    base = wid * PER_W

    pltpu.sync_copy(tok_hbm.at[pl.ds(base, PER_W)], tok_idx)
    pltpu.sync_copy(cid_hbm.at[pl.ds(base, PER_W)], cid_idx)
    plsc.subcore_barrier()

    def fire_in(cg, b):
        tvec = tok_idx[pl.ds(cg * C, C)]
        cvec = cid_idx[pl.ds(cg * C, C)]
        pltpu.async_copy(ttab.at[tvec], bufT[b], sem_in[b])
        pltpu.async_copy(ctab.at[cvec], bufC[b], sem_in[b])

    def drain_in(b):
        pltpu.make_async_copy(ttab.at[pl.ds(0, C)], bufT[b], sem_in[b]).wait()
        pltpu.make_async_copy(ctab.at[pl.ds(0, C)], bufC[b], sem_in[b]).wait()

    def fire_out(cg, b):
        pltpu.async_copy(bufO[b], out_hbm.at[pl.ds(base + cg * C, C)], sem_out[b])

    def drain_out(b):
        pltpu.make_async_copy(
            bufO[b], out_hbm.at[pl.ds(0, C)], sem_out[b]).wait()

    fire_in(0, 0)
    fire_in(1, 1)

    def step(q, carry):
        for b in range(NBUF):
            cg = q * NBUF + b
            drain_in(b)

            @pl.when(cg >= 2)
            def _():
                drain_out(b)

            def add_row(t, carry2):
                for k2 in range(DV // 2):
                    x32 = bufC[b][t, pl.ds(k2 * LANES, LANES)]
                    x = plsc.bitcast(x32, jnp.bfloat16)
                    lo, hi = plsc.unpack(x, format=plsc.PackFormat.INTERLEAVED)
                    sl0 = pl.ds(k2 * 2 * LANES, LANES)
                    sl1 = pl.ds(k2 * 2 * LANES + LANES, LANES)
                    bufO[b][t, sl0] = bufT[b][t, sl0] + lo
                    bufO[b][t, sl1] = bufT[b][t, sl1] + hi
                return carry2

            lax.fori_loop(0, C, add_row, 0)
            fire_out(cg, b)

            @pl.when(cg + 2 < NCHUNK)
            def _():
                fire_in(cg + 2, b)
        return carry

    lax.fori_loop(0, NCHUNK // NBUF, step, 0)

    drain_out(0)
    drain_out(1)


@jax.jit
def _embed_sum(tok, cid, ttab, ctab):
    mesh = plsc.VectorSubcoreMesh(core_axis_name="c", subcore_axis_name="s")
    scratch = [
        pltpu.VMEM((PER_W,), jnp.int32),
        pltpu.VMEM((PER_W,), jnp.int32),
    ]
    scratch += [pltpu.VMEM((C, D), jnp.float32) for _ in range(NBUF)]
    scratch += [pltpu.VMEM((C, D // 2), jnp.int32) for _ in range(NBUF)]
    scratch += [pltpu.VMEM((C, D), jnp.float32) for _ in range(NBUF)]
    scratch += [pltpu.SemaphoreType.DMA for _ in range(2 * NBUF)]
    f = functools.partial(
        pl.kernel,
        mesh=mesh,
        out_type=jax.ShapeDtypeStruct((N, D), jnp.float32),
        scratch_types=scratch,
        compiler_params=pltpu.CompilerParams(needs_layout_passes=False),
    )(_sc_body)
    return f(tok, cid, ttab, ctab)


def kernel(token_ids, position_ids, segment_ids, token_table, position_table, segment_table):
    comb, cid = _prep(position_table, segment_table,
                      position_ids.astype(jnp.int32), segment_ids.astype(jnp.int32))
    comb_bf = jax.lax.bitcast_convert_type(
        comb.reshape(1024, D // 32, 2, 16)
        .transpose(0, 1, 3, 2)
        .reshape(1024, D)
        .astype(jnp.bfloat16)
        .reshape(1024, D // 2, 2),
        jnp.int32).reshape(1024, D // 2)
    tok = token_ids.reshape(N).astype(jnp.int32)
    out = _embed_sum(tok, cid.reshape(N), token_table, comb_bf)
    return out.reshape(B, L, D)


# speedup vs baseline: 3.5682x; 1.0003x over previous
import functools

import jax
import jax.numpy as jnp
from jax import lax
from jax.experimental import pallas as pl
from jax.experimental.pallas import tpu as pltpu
from jax.experimental.pallas import tpu_sc as plsc

B, L, D = 128, 512, 768
N = B * L
NC, NS, LANES = 2, 16, 16
NW = NC * NS
PER_W = N // NW
C = LANES
NCHUNK = PER_W // C
NBUF = 2
DV = D // LANES


def _prep_body(ptab, stab, pos, seg, comb, cid):
    p = ptab[...]
    comb[pl.ds(0, 512), :] = p + stab[0:1, :]
    comb[pl.ds(512, 512), :] = p + stab[1:2, :]
    cid[...] = seg[...] * 512 + pos[...]


@jax.jit
def _prep(ptab, stab, pos, seg):
    return pl.pallas_call(
        _prep_body,
        out_shape=(
            jax.ShapeDtypeStruct((2 * 512, D), jnp.float32),
            jax.ShapeDtypeStruct((B, L), jnp.int32),
        ),
    )(ptab, stab, pos, seg)


def _sc_body(tok_hbm, cid_hbm, ttab, ctab, out_hbm, *scratch):
    tok_idx, cid_idx = scratch[0], scratch[1]
    bufT = scratch[2:2 + NBUF]
    bufC = scratch[2 + NBUF:2 + 2 * NBUF]
    bufO = scratch[2 + 2 * NBUF:2 + 3 * NBUF]
    sem_in = scratch[2 + 3 * NBUF:2 + 4 * NBUF]
    sem_out = scratch[2 + 4 * NBUF:2 + 5 * NBUF]

    sid = lax.axis_index("s")
    wid = sid * NC + lax.axis_index("c")
    base = wid * PER_W

    pltpu.sync_copy(tok_hbm.at[pl.ds(base, PER_W)], tok_idx)
    pltpu.sync_copy(cid_hbm.at[pl.ds(base, PER_W)], cid_idx)
    plsc.subcore_barrier()

    def fire_in(cg, b):
        tvec = tok_idx[pl.ds(cg * C, C)]
        cvec = cid_idx[pl.ds(cg * C, C)]
        pltpu.async_copy(ttab.at[tvec], bufT[b], sem_in[b])
        pltpu.async_copy(ctab.at[cvec], bufC[b], sem_in[b])

    def drain_in(b):
        pltpu.make_async_copy(ttab.at[pl.ds(0, C)], bufT[b], sem_in[b]).wait()
        pltpu.make_async_copy(ctab.at[pl.ds(0, C)], bufC[b], sem_in[b]).wait()

    def fire_out(cg, b):
        pltpu.async_copy(bufO[b], out_hbm.at[pl.ds(base + cg * C, C)], sem_out[b])

    def drain_out(b):
        pltpu.make_async_copy(
            bufO[b], out_hbm.at[pl.ds(0, C)], sem_out[b]).wait()

    fire_in(0, 0)
    fire_in(1, 1)

    def step(q, carry):
        for b in range(NBUF):
            cg = q * NBUF + b
            drain_in(b)

            @pl.when(cg >= 2)
            def _():
                drain_out(b)

            def add_row(t, carry2):
                for k2 in range(DV // 2):
                    x32 = bufC[b][t, pl.ds(k2 * LANES, LANES)]
                    lo = lax.bitcast_convert_type(x32 << 16, jnp.float32)
                    hi = lax.bitcast_convert_type(x32 & jnp.int32(-65536), jnp.float32)
                    sl0 = pl.ds(k2 * 2 * LANES, LANES)
                    sl1 = pl.ds(k2 * 2 * LANES + LANES, LANES)
                    bufO[b][t, sl0] = bufT[b][t, sl0] + lo
                    bufO[b][t, sl1] = bufT[b][t, sl1] + hi
                return carry2

            lax.fori_loop(0, C, add_row, 0)
            fire_out(cg, b)

            @pl.when(cg + 2 < NCHUNK)
            def _():
                fire_in(cg + 2, b)
        return carry

    lax.fori_loop(0, NCHUNK // NBUF, step, 0)

    drain_out(0)
    drain_out(1)


@jax.jit
def _embed_sum(tok, cid, ttab, ctab):
    mesh = plsc.VectorSubcoreMesh(core_axis_name="c", subcore_axis_name="s")
    scratch = [
        pltpu.VMEM((PER_W,), jnp.int32),
        pltpu.VMEM((PER_W,), jnp.int32),
    ]
    scratch += [pltpu.VMEM((C, D), jnp.float32) for _ in range(NBUF)]
    scratch += [pltpu.VMEM((C, D // 2), jnp.int32) for _ in range(NBUF)]
    scratch += [pltpu.VMEM((C, D), jnp.float32) for _ in range(NBUF)]
    scratch += [pltpu.SemaphoreType.DMA for _ in range(2 * NBUF)]
    f = functools.partial(
        pl.kernel,
        mesh=mesh,
        out_type=jax.ShapeDtypeStruct((N, D), jnp.float32),
        scratch_types=scratch,
    )(_sc_body)
    return f(tok, cid, ttab, ctab)


def kernel(token_ids, position_ids, segment_ids, token_table, position_table, segment_table):
    comb, cid = _prep(position_table, segment_table,
                      position_ids.astype(jnp.int32), segment_ids.astype(jnp.int32))
    comb_bf = jax.lax.bitcast_convert_type(
        comb.reshape(1024, D // 32, 2, 16)
        .transpose(0, 1, 3, 2)
        .reshape(1024, D)
        .astype(jnp.bfloat16)
        .reshape(1024, D // 2, 2),
        jnp.int32).reshape(1024, D // 2)
    tok = token_ids.reshape(N).astype(jnp.int32)
    out = _embed_sum(tok, cid.reshape(N), token_table, comb_bf)
    return out.reshape(B, L, D)


# speedup vs baseline: 4.6706x; 1.3090x over previous
import functools

import jax
import jax.numpy as jnp
from jax import lax
from jax.experimental import pallas as pl
from jax.experimental.pallas import tpu as pltpu
from jax.experimental.pallas import tpu_sc as plsc

B, L, D = 128, 512, 768
N = B * L
NC, NS, LANES = 2, 16, 16
NW = NC * NS
PER_W = N // NW
C = LANES
NCHUNK = PER_W // C
NBUF = 2
DV = D // LANES


def _prep_body(ptab, stab, pos, seg, comb, cid):
    p = ptab[...]
    comb[pl.ds(0, 512), :] = p + stab[0:1, :]
    comb[pl.ds(512, 512), :] = p + stab[1:2, :]
    cid[...] = seg[...] * 512 + pos[...]


@jax.jit
def _prep(ptab, stab, pos, seg):
    return pl.pallas_call(
        _prep_body,
        out_shape=(
            jax.ShapeDtypeStruct((2 * 512, D), jnp.float32),
            jax.ShapeDtypeStruct((B, L), jnp.int32),
        ),
    )(ptab, stab, pos, seg)


def _sc_body(tok_hbm, cid_hbm, ttab, ctab, out_hbm, *scratch):
    tok_idx, cid_idx = scratch[0], scratch[1]
    bufT = scratch[2:2 + NBUF]
    bufC = scratch[2 + NBUF:2 + 2 * NBUF]
    bufO = scratch[2 + 2 * NBUF:2 + 3 * NBUF]
    sem_in = scratch[2 + 3 * NBUF:2 + 4 * NBUF]
    sem_out = scratch[2 + 4 * NBUF:2 + 5 * NBUF]

    sid = lax.axis_index("s")
    wid = sid * NC + lax.axis_index("c")
    base = wid * PER_W

    pltpu.sync_copy(tok_hbm.at[pl.ds(base, PER_W)], tok_idx)
    pltpu.sync_copy(cid_hbm.at[pl.ds(base, PER_W)], cid_idx)
    plsc.subcore_barrier()

    def fire_in(cg, b):
        tvec = tok_idx[pl.ds(cg * C, C)]
        cvec = cid_idx[pl.ds(cg * C, C)]
        pltpu.async_copy(ttab.at[tvec], bufT[b], sem_in[b])
        pltpu.async_copy(ctab.at[cvec], bufC[b], sem_in[b])

    def drain_in(b):
        pltpu.make_async_copy(ttab.at[pl.ds(0, C)], bufT[b], sem_in[b]).wait()
        pltpu.make_async_copy(ttab.at[pl.ds(0, C)], bufC[b], sem_in[b]).wait()

    def fire_out(cg, b):
        pltpu.async_copy(bufO[b], out_hbm.at[pl.ds(base + cg * C, C)], sem_out[b])

    def drain_out(b):
        pltpu.make_async_copy(
            bufO[b], out_hbm.at[pl.ds(0, C)], sem_out[b]).wait()

    fire_in(0, 0)
    fire_in(1, 1)

    def step(q, carry):
        for b in range(NBUF):
            cg = q * NBUF + b
            drain_in(b)

            @pl.when(cg >= 2)
            def _():
                drain_out(b)

            def add_row(t, carry2):
                for k in range(DV):
                    sl = pl.ds(k * LANES, LANES)
                    bufO[b][t, sl] = bufT[b][t, sl] + bufC[b][t, sl]
                return carry2

            lax.fori_loop(0, C, add_row, 0)
            fire_out(cg, b)

            @pl.when(cg + 2 < NCHUNK)
            def _():
                fire_in(cg + 2, b)
        return carry

    lax.fori_loop(0, NCHUNK // NBUF, step, 0)

    drain_out(0)
    drain_out(1)


@jax.jit
def _embed_sum(tok, cid, ttab, ctab):
    mesh = plsc.VectorSubcoreMesh(core_axis_name="c", subcore_axis_name="s")
    scratch = [
        pltpu.VMEM((PER_W,), jnp.int32),
        pltpu.VMEM((PER_W,), jnp.int32),
    ]
    scratch += [pltpu.VMEM((C, D), jnp.float32) for _ in range(3 * NBUF)]
    scratch += [pltpu.SemaphoreType.DMA for _ in range(2 * NBUF)]
    f = functools.partial(
        pl.kernel,
        mesh=mesh,
        out_type=jax.ShapeDtypeStruct((N, D), jnp.float32),
        scratch_types=scratch,
    )(_sc_body)
    return f(tok, cid, ttab, ctab)


def kernel(token_ids, position_ids, segment_ids, token_table, position_table, segment_table):
    comb, cid = _prep(position_table, segment_table,
                      position_ids.astype(jnp.int32), segment_ids.astype(jnp.int32))
    tok = token_ids.reshape(N).astype(jnp.int32)
    out = _embed_sum(tok, cid.reshape(N), token_table, comb)
    return out.reshape(B, L, D)
